# baseline (speedup 1.0000x reference)
import functools

import jax
import jax.numpy as jnp
from jax import lax
from jax.experimental import pallas as pl
from jax.experimental.pallas import tpu as pltpu
from jax.experimental.pallas import tpu_sc as plsc

N_POS = 131072
N_BITS = 13
N_OUT = 128
RAM_SIZE = 1 << N_BITS

_HALF = N_OUT // 2
_NEUR_PER_TILE = _HALF // 16
_NEUR_GROUP = 2
_SUP = 1
_CHUNK = _SUP * 128


def _body(pos_hbm, ram_hbm, conn_hbm, out_hbm,
          conn_v, ram_row, buf2d, t_sh, idx_all, rows_v, rows_v2,
          sem_g, sem_w, sem_w2):
    cid = lax.axis_index("c")
    sid = lax.axis_index("s")
    lanes = lax.iota(jnp.int32, 16)

    n_base = cid * _HALF + sid * _NEUR_PER_TILE
    pltpu.sync_copy(conn_hbm.at[pl.ds(n_base, _NEUR_PER_TILE)], conn_v)

    def neuron_body(nk, carry):
        k = nk % _NEUR_GROUP
        n = n_base + nk
        pltpu.sync_copy(ram_hbm.at[n], ram_row)
        conn_vec = conn_v[nk]
        cs = [conn_vec[j] for j in range(N_BITS)]

        def sigma(p):
            a = jnp.zeros_like(p)
            for j in range(N_BITS):
                a = a + (((p >> (12 - cs[j])) & 1) << (12 - j))
            return a

        addr_lo = sigma(lanes)
        kvec = jnp.full((16,), k, jnp.int32)

        def group_body(g, c):
            hi_vec = sigma((lanes + g * 16) << 4)
            for l in range(16):
                idx = addr_lo + hi_vec[l]
                vals = plsc.load_gather(ram_row, [idx])
                rows = (g * 16 + l) * 16 + lanes
                plsc.store_scatter(buf2d, [rows, kvec], vals)
            return c
        lax.fori_loop(0, RAM_SIZE // 256, group_body, 0)
        return carry

    def pass_body(h, carry):
        lax.fori_loop(h * _NEUR_GROUP, (h + 1) * _NEUR_GROUP, neuron_body, 0)
        pltpu.sync_copy(
            buf2d,
            t_sh.at[:, pl.ds(sid * _NEUR_PER_TILE + h * _NEUR_GROUP, _NEUR_GROUP)])
        return carry

    lax.fori_loop(0, _NEUR_PER_TILE // _NEUR_GROUP, pass_body, 0)
    plsc.subcore_barrier()

    rows_per_tile = (N_POS // 128) // 16
    half_rows = rows_per_tile // 2
    chunks_per_half = half_rows // _SUP
    col0 = cid * _HALF
    base_row = sid * rows_per_tile
    rows_bufs = (rows_v, rows_v2)
    wb_sems = (sem_w, sem_w2)
    _NB = len(rows_bufs)

    def drain(b):
        pltpu.make_async_copy(
            out_hbm.at[pl.ds(0, _CHUNK), pl.ds(0, _HALF)],
            rows_bufs[b], wb_sems[b]).wait()

    def do_chunk(git, lit, b, need_guard):
        rows_b = rows_bufs[b]
        cps = [
            pltpu.async_copy(
                t_sh.at[idx_all.at[lit * _SUP + r]],
                rows_b.at[pl.ds(r * 128, 128)], sem_g)
            for r in range(_SUP)
        ]
        for c in cps:
            c.wait()

        nxt = (b + 1) % _NB
        if need_guard:
            @pl.when(git >= _NB - 1)
            def _():
                drain(nxt)
        else:
            drain(nxt)

        row0 = base_row + git * _SUP
        pltpu.async_copy(
            rows_b, out_hbm.at[pl.ds(row0 * 128, _CHUNK), pl.ds(col0, _HALF)],
            wb_sems[b])

    main_iters = (chunks_per_half - chunks_per_half % _NB) // _NB
    for hh in (0, 1):
        pltpu.sync_copy(pos_hbm.at[pl.ds(base_row + hh * half_rows, half_rows)],
                        idx_all)
        gbase = hh * chunks_per_half

        def outer(o, c, hh=hh, gbase=gbase):
            for j in range(_NB):
                lit = _NB * o + j
                do_chunk(gbase + lit, lit, (gbase + j) % _NB,
                         need_guard=(hh == 0))
            return c

        lax.fori_loop(0, main_iters, outer, 0)
        for lit in range(main_iters * _NB, chunks_per_half):
            git = gbase + lit
            do_chunk(git, lit, git % _NB, need_guard=False)

    last = 2 * chunks_per_half - 1
    for g in range(last - _NB + 2, last + 1):
        drain(g % _NB)


@jax.jit
def _sc_call(pos2, ram_memory, conn_flat):
    mesh = plsc.VectorSubcoreMesh(core_axis_name="c", subcore_axis_name="s")
    fn = pl.kernel(
        _body,
        out_type=jax.ShapeDtypeStruct((N_POS, N_OUT), jnp.float32),
        mesh=mesh,
        compiler_params=pltpu.CompilerParams(
            needs_layout_passes=False, use_tc_tiling_on_sc=False),
        scratch_types=[
            pltpu.VMEM((_NEUR_PER_TILE, 16), jnp.int32),
            pltpu.VMEM((RAM_SIZE,), jnp.float32),
            pltpu.VMEM((RAM_SIZE, _NEUR_GROUP), jnp.float32),
            pltpu.VMEM_SHARED((RAM_SIZE, _HALF), jnp.float32),
            pltpu.VMEM((32, 128), jnp.int32),
            pltpu.VMEM((_CHUNK, _HALF), jnp.float32),
            pltpu.VMEM((_CHUNK, _HALF), jnp.float32),
            pltpu.SemaphoreType.DMA,
            pltpu.SemaphoreType.DMA,
            pltpu.SemaphoreType.DMA,
        ],
    )
    return fn(pos2, ram_memory, conn_flat)


def kernel(positions, ram_memory, conn_map):
    pos2 = positions.reshape(N_POS // 128, 128)
    conn_pad = jnp.pad(conn_map, ((0, 0), (0, 16 - N_BITS)))
    return _sc_call(pos2, ram_memory, conn_pad)


# speedup vs baseline: 1.0118x; 1.0118x over previous
import functools

import jax
import jax.numpy as jnp
from jax import lax
from jax.experimental import pallas as pl
from jax.experimental.pallas import tpu as pltpu
from jax.experimental.pallas import tpu_sc as plsc

N_POS = 131072
N_BITS = 13
N_OUT = 128
RAM_SIZE = 1 << N_BITS

_HALF = N_OUT // 2
_NEUR_PER_TILE = _HALF // 16
_NEUR_GROUP = 2
_SUP = 1
_CHUNK = _SUP * 128


def _body(pos_hbm, ram_hbm, conn_hbm, out_hbm,
          conn_v, ram_row, buf2d, t_sh, idx_all, rows_v, rows_v2,
          sem_g, sem_g2, sem_w, sem_w2):
    cid = lax.axis_index("c")
    sid = lax.axis_index("s")
    lanes = lax.iota(jnp.int32, 16)

    n_base = cid * _HALF + sid * _NEUR_PER_TILE
    pltpu.sync_copy(conn_hbm.at[pl.ds(n_base, _NEUR_PER_TILE)], conn_v)

    def neuron_body(nk, carry):
        k = nk % _NEUR_GROUP
        n = n_base + nk
        pltpu.sync_copy(ram_hbm.at[n], ram_row)
        conn_vec = conn_v[nk]
        cs = [conn_vec[j] for j in range(N_BITS)]

        def sigma(p):
            a = jnp.zeros_like(p)
            for j in range(N_BITS):
                a = a + (((p >> (12 - cs[j])) & 1) << (12 - j))
            return a

        addr_lo = sigma(lanes)
        kvec = jnp.full((16,), k, jnp.int32)

        def group_body(g, c):
            hi_vec = sigma((lanes + g * 16) << 4)
            for l in range(16):
                idx = addr_lo + hi_vec[l]
                vals = plsc.load_gather(ram_row, [idx])
                rows = (g * 16 + l) * 16 + lanes
                plsc.store_scatter(buf2d, [rows, kvec], vals)
            return c
        lax.fori_loop(0, RAM_SIZE // 256, group_body, 0)
        return carry

    def pass_body(h, carry):
        lax.fori_loop(h * _NEUR_GROUP, (h + 1) * _NEUR_GROUP, neuron_body, 0)
        pltpu.sync_copy(
            buf2d,
            t_sh.at[:, pl.ds(sid * _NEUR_PER_TILE + h * _NEUR_GROUP, _NEUR_GROUP)])
        return carry

    lax.fori_loop(0, _NEUR_PER_TILE // _NEUR_GROUP, pass_body, 0)
    plsc.subcore_barrier()

    rows_per_tile = (N_POS // 128) // 16
    half_rows = rows_per_tile // 2
    chunks_per_half = half_rows // _SUP
    col0 = cid * _HALF
    base_row = sid * rows_per_tile
    rows_bufs = (rows_v, rows_v2)
    wb_sems = (sem_w, sem_w2)
    g_sems = (sem_g, sem_g2)

    def drain_wb(b):
        pltpu.make_async_copy(
            out_hbm.at[pl.ds(0, _CHUNK), pl.ds(0, _HALF)],
            rows_bufs[b], wb_sems[b]).wait()

    def fire_gather(lit, b):
        pltpu.async_copy(t_sh.at[idx_all.at[lit]], rows_bufs[b], g_sems[b])

    def wait_gather(b):
        pltpu.make_async_copy(
            t_sh.at[idx_all.at[0]], rows_bufs[b], g_sems[b]).wait()

    def finish_chunk(git, b):
        wait_gather(b)
        pltpu.async_copy(
            rows_bufs[b],
            out_hbm.at[pl.ds((base_row + git) * 128, _CHUNK),
                       pl.ds(col0, _HALF)],
            wb_sems[b])

    for hh in (0, 1):
        pltpu.sync_copy(pos_hbm.at[pl.ds(base_row + hh * half_rows, half_rows)],
                        idx_all)
        gbase = hh * chunks_per_half
        if hh == 1:
            drain_wb(0)
        fire_gather(0, 0)
        if hh == 1:
            drain_wb(1)
        fire_gather(1, 1)
        finish_chunk(gbase, 0)

        def steady(o, c, gbase=gbase):
            for j in (0, 1):
                lit = 1 + 2 * o + j
                b = (1 + j) % 2
                nxt = (b + 1) % 2
                drain_wb(nxt)
                fire_gather(lit + 1, nxt)
                finish_chunk(gbase + lit, b)
            return c

        lax.fori_loop(0, (chunks_per_half - 4) // 2, steady, 0)
        for lit in (chunks_per_half - 3, chunks_per_half - 2):
            b = lit % 2
            nxt = (b + 1) % 2
            drain_wb(nxt)
            fire_gather(lit + 1, nxt)
            finish_chunk(gbase + lit, b)
        finish_chunk(gbase + chunks_per_half - 1, (chunks_per_half - 1) % 2)

    drain_wb(0)
    drain_wb(1)


@jax.jit
def _sc_call(pos2, ram_memory, conn_flat):
    mesh = plsc.VectorSubcoreMesh(core_axis_name="c", subcore_axis_name="s")
    fn = pl.kernel(
        _body,
        out_type=jax.ShapeDtypeStruct((N_POS, N_OUT), jnp.float32),
        mesh=mesh,
        compiler_params=pltpu.CompilerParams(
            needs_layout_passes=False, use_tc_tiling_on_sc=False),
        scratch_types=[
            pltpu.VMEM((_NEUR_PER_TILE, 16), jnp.int32),
            pltpu.VMEM((RAM_SIZE,), jnp.float32),
            pltpu.VMEM((RAM_SIZE, _NEUR_GROUP), jnp.float32),
            pltpu.VMEM_SHARED((RAM_SIZE, _HALF), jnp.float32),
            pltpu.VMEM((32, 128), jnp.int32),
            pltpu.VMEM((_CHUNK, _HALF), jnp.float32),
            pltpu.VMEM((_CHUNK, _HALF), jnp.float32),
            pltpu.SemaphoreType.DMA,
            pltpu.SemaphoreType.DMA,
            pltpu.SemaphoreType.DMA,
            pltpu.SemaphoreType.DMA,
        ],
    )
    return fn(pos2, ram_memory, conn_flat)


def kernel(positions, ram_memory, conn_map):
    pos2 = positions.reshape(N_POS // 128, 128)
    conn_pad = jnp.pad(conn_map, ((0, 0), (0, 16 - N_BITS)))
    return _sc_call(pos2, ram_memory, conn_pad)


# speedup vs baseline: 1.0163x; 1.0044x over previous
import functools

import jax
import jax.numpy as jnp
from jax import lax
from jax.experimental import pallas as pl
from jax.experimental.pallas import tpu as pltpu
from jax.experimental.pallas import tpu_sc as plsc

N_POS = 131072
N_BITS = 13
N_OUT = 128
RAM_SIZE = 1 << N_BITS

_HALF = N_OUT // 2
_NEUR_PER_TILE = _HALF // 16
_NEUR_GROUP = 2
_SUP = 1
_CHUNK = _SUP * 128


def _body(pos_hbm, ram_hbm, conn_hbm, out_hbm,
          conn_v, ram_row, buf2d, t_sh, idx_all, rows_v, rows_v2,
          sem_g, sem_g2, sem_w, sem_w2):
    cid = lax.axis_index("c")
    sid = lax.axis_index("s")
    lanes = lax.iota(jnp.int32, 16)

    n_base = cid * _HALF + sid * _NEUR_PER_TILE
    pltpu.sync_copy(conn_hbm.at[pl.ds(n_base, _NEUR_PER_TILE)], conn_v)

    _pre_base = sid * ((N_POS // 128) // 16)
    pltpu.async_copy(pos_hbm.at[pl.ds(_pre_base, 32)], idx_all, sem_g)

    def neuron_body(nk, carry):
        k = nk % _NEUR_GROUP
        n = n_base + nk
        pltpu.sync_copy(ram_hbm.at[n], ram_row)
        conn_vec = conn_v[nk]
        cs = [conn_vec[j] for j in range(N_BITS)]

        def sigma(p):
            a = jnp.zeros_like(p)
            for j in range(N_BITS):
                a = a + (((p >> (12 - cs[j])) & 1) << (12 - j))
            return a

        addr_lo = sigma(lanes)
        kvec = jnp.full((16,), k, jnp.int32)

        def group_body(g, c):
            hi_vec = sigma((lanes + g * 16) << 4)
            for l in range(16):
                idx = addr_lo + hi_vec[l]
                vals = plsc.load_gather(ram_row, [idx])
                rows = (g * 16 + l) * 16 + lanes
                plsc.store_scatter(buf2d, [rows, kvec], vals)
            return c
        lax.fori_loop(0, RAM_SIZE // 256, group_body, 0)
        return carry

    def pass_body(h, carry):
        lax.fori_loop(h * _NEUR_GROUP, (h + 1) * _NEUR_GROUP, neuron_body, 0)
        pltpu.sync_copy(
            buf2d,
            t_sh.at[:, pl.ds(sid * _NEUR_PER_TILE + h * _NEUR_GROUP, _NEUR_GROUP)])
        return carry

    lax.fori_loop(0, _NEUR_PER_TILE // _NEUR_GROUP, pass_body, 0)
    plsc.subcore_barrier()

    rows_per_tile = (N_POS // 128) // 16
    half_rows = rows_per_tile // 2
    chunks_per_half = half_rows // _SUP
    col0 = cid * _HALF
    base_row = sid * rows_per_tile
    rows_bufs = (rows_v, rows_v2)
    wb_sems = (sem_w, sem_w2)
    g_sems = (sem_g, sem_g2)

    def drain_wb(b):
        pltpu.make_async_copy(
            out_hbm.at[pl.ds(0, _CHUNK), pl.ds(0, _HALF)],
            rows_bufs[b], wb_sems[b]).wait()

    def fire_gather(lit, b):
        pltpu.async_copy(t_sh.at[idx_all.at[lit]], rows_bufs[b], g_sems[b])

    def wait_gather(b):
        pltpu.make_async_copy(
            t_sh.at[idx_all.at[0]], rows_bufs[b], g_sems[b]).wait()

    def finish_chunk(git, b):
        wait_gather(b)
        pltpu.async_copy(
            rows_bufs[b],
            out_hbm.at[pl.ds((base_row + git) * 128, _CHUNK),
                       pl.ds(col0, _HALF)],
            wb_sems[b])

    for hh in (0, 1):
        if hh == 0:
            pltpu.make_async_copy(
                pos_hbm.at[pl.ds(base_row, half_rows)], idx_all, sem_g).wait()
        else:
            pltpu.sync_copy(
                pos_hbm.at[pl.ds(base_row + hh * half_rows, half_rows)],
                idx_all)
        gbase = hh * chunks_per_half
        if hh == 1:
            drain_wb(0)
        fire_gather(0, 0)
        if hh == 1:
            drain_wb(1)
        fire_gather(1, 1)
        finish_chunk(gbase, 0)

        def steady(o, c, gbase=gbase):
            for j in (0, 1):
                lit = 1 + 2 * o + j
                b = (1 + j) % 2
                nxt = (b + 1) % 2
                drain_wb(nxt)
                fire_gather(lit + 1, nxt)
                finish_chunk(gbase + lit, b)
            return c

        lax.fori_loop(0, (chunks_per_half - 4) // 2, steady, 0)
        for lit in (chunks_per_half - 3, chunks_per_half - 2):
            b = lit % 2
            nxt = (b + 1) % 2
            drain_wb(nxt)
            fire_gather(lit + 1, nxt)
            finish_chunk(gbase + lit, b)
        finish_chunk(gbase + chunks_per_half - 1, (chunks_per_half - 1) % 2)

    drain_wb(0)
    drain_wb(1)


@jax.jit
def _sc_call(pos2, ram_memory, conn_flat):
    mesh = plsc.VectorSubcoreMesh(core_axis_name="c", subcore_axis_name="s")
    fn = pl.kernel(
        _body,
        out_type=jax.ShapeDtypeStruct((N_POS, N_OUT), jnp.float32),
        mesh=mesh,
        compiler_params=pltpu.CompilerParams(
            needs_layout_passes=False, use_tc_tiling_on_sc=False),
        scratch_types=[
            pltpu.VMEM((_NEUR_PER_TILE, 16), jnp.int32),
            pltpu.VMEM((RAM_SIZE,), jnp.float32),
            pltpu.VMEM((RAM_SIZE, _NEUR_GROUP), jnp.float32),
            pltpu.VMEM_SHARED((RAM_SIZE, _HALF), jnp.float32),
            pltpu.VMEM((32, 128), jnp.int32),
            pltpu.VMEM((_CHUNK, _HALF), jnp.float32),
            pltpu.VMEM((_CHUNK, _HALF), jnp.float32),
            pltpu.SemaphoreType.DMA,
            pltpu.SemaphoreType.DMA,
            pltpu.SemaphoreType.DMA,
            pltpu.SemaphoreType.DMA,
        ],
    )
    return fn(pos2, ram_memory, conn_flat)


def kernel(positions, ram_memory, conn_map):
    pos2 = positions.reshape(N_POS // 128, 128)
    conn_pad = jnp.pad(conn_map, ((0, 0), (0, 16 - N_BITS)))
    return _sc_call(pos2, ram_memory, conn_pad)
